# baseline (speedup 1.0000x reference)
import jax
import jax.numpy as jnp
from jax import lax
from jax.experimental import pallas as pl
from jax.experimental.pallas import tpu as pltpu

_MAP = 100
_WIN = 11
_EGO = 33
_NCLS = 4
_CHAN = _NCLS + 2
_LSTM_IN = _WIN * _WIN * _CHAN
_IN_PAD = 768
_HPAD = 1024
_NH = 512
_M = 9
_K0 = _IN_PAD + _HPAD
_OUT = _WIN * _WIN * _NCLS
_VMEM = 64 * 1024 * 1024


def _lstm2_kernel(xh0_ref, w0_ref, b0_ref, c0p_ref, hp1_ref, w1a_ref, w1b_ref,
                  h0_ref, c0_ref, part_ref, ig_scr):
    s = pl.program_id(1)

    @pl.when(s < 4)
    def _layer0():
        pre = jnp.dot(xh0_ref[...].astype(jnp.bfloat16), w0_ref[0],
                      preferred_element_type=jnp.float32) + b0_ref[...]

        @pl.when(s == 0)
        def _():
            ig_scr[...] = jax.nn.sigmoid(pre)

        @pl.when(s == 1)
        def _():
            c0_ref[...] = jax.nn.sigmoid(pre) * c0p_ref[...]

        @pl.when(s == 2)
        def _():
            c0_ref[...] = c0_ref[...] + ig_scr[...] * jnp.tanh(pre)

        @pl.when(s == 3)
        def _():
            h0_ref[...] = jax.nn.sigmoid(pre) * jnp.tanh(c0_ref[...])

    @pl.when(s >= 4)
    def _layer1_partial():
        part_ref[0] = (
            jnp.dot(h0_ref[...].astype(jnp.bfloat16), w1a_ref[0],
                    preferred_element_type=jnp.float32)
            + jnp.dot(hp1_ref[...].astype(jnp.bfloat16), w1b_ref[0],
                      preferred_element_type=jnp.float32))


def _run_lstm_pair(xh0, c0_prev, h1_prev, w0, b0, w1):
    return pl.pallas_call(
        _lstm2_kernel,
        out_shape=(
            jax.ShapeDtypeStruct((_M, _HPAD), jnp.float32),
            jax.ShapeDtypeStruct((_M, _HPAD), jnp.float32),
            jax.ShapeDtypeStruct((2, _M, 8 * _NH), jnp.float32),
        ),
        grid_spec=pltpu.PrefetchScalarGridSpec(
            num_scalar_prefetch=0,
            grid=(2, 12),
            in_specs=[
                pl.BlockSpec((_M, _K0), lambda n, s: (0, 0)),
                pl.BlockSpec((1, _K0, _NH),
                             lambda n, s: (jnp.minimum(s, 3) * 2 + n, 0, 0)),
                pl.BlockSpec((1, _NH),
                             lambda n, s: (0, jnp.minimum(s, 3) * 2 + n)),
                pl.BlockSpec((_M, _NH), lambda n, s: (0, n)),
                pl.BlockSpec((_M, _NH), lambda n, s: (0, n)),
                pl.BlockSpec((1, _NH, _NH),
                             lambda n, s: (jnp.maximum(s - 4, 0), n, 0)),
                pl.BlockSpec((1, _NH, _NH),
                             lambda n, s: (jnp.maximum(s - 4, 0), 2 + n, 0)),
            ],
            out_specs=(
                pl.BlockSpec((_M, _NH), lambda n, s: (0, n)),
                pl.BlockSpec((_M, _NH), lambda n, s: (0, n)),
                pl.BlockSpec((1, _M, _NH),
                             lambda n, s: (n, 0, jnp.maximum(s - 4, 0))),
            ),
            scratch_shapes=[pltpu.VMEM((_M, _NH), jnp.float32)],
        ),
        compiler_params=pltpu.CompilerParams(
            dimension_semantics=("parallel", "arbitrary"),
            vmem_limit_bytes=_VMEM,
        ),
    )(xh0, w0, b0, c0_prev, h1_prev, w1, w1)


def _head_kernel(pa_ref, pb_ref, b1_ref, c1p_ref, h0_ref, c0_ref,
                 w1_ref, bf1_ref, w2_ref, bf2_ref, w3_ref, bf3_ref,
                 out_ref, hs_ref, cs_ref):
    pre = pa_ref[0] + pb_ref[0] + b1_ref[...]
    gi = jax.nn.sigmoid(pre[:, 0 * _HPAD:1 * _HPAD])
    gf = jax.nn.sigmoid(pre[:, 1 * _HPAD:2 * _HPAD])
    gg = jnp.tanh(pre[:, 2 * _HPAD:3 * _HPAD])
    go = jax.nn.sigmoid(pre[:, 3 * _HPAD:4 * _HPAD])
    c1 = gf * c1p_ref[...] + gi * gg
    h1 = go * jnp.tanh(c1)
    hs_ref[0] = h0_ref[...]
    hs_ref[1] = h1
    cs_ref[0] = c0_ref[...]
    cs_ref[1] = c1
    t = jnp.dot(h1.astype(jnp.bfloat16), w1_ref[...],
                preferred_element_type=jnp.float32) + bf1_ref[...]
    t = jnp.maximum(t, 0.0)
    t = jnp.dot(t.astype(jnp.bfloat16), w2_ref[...],
                preferred_element_type=jnp.float32) + bf2_ref[...]
    t = jnp.maximum(t, 0.0)
    out_ref[...] = jnp.dot(t.astype(jnp.bfloat16), w3_ref[...],
                           preferred_element_type=jnp.float32) + bf3_ref[...]


def _run_head(part, b1, c1_prev, h0, c0, w1, bf1, w2, bf2, w3, bf3):
    operands = (part, part, b1, c1_prev, h0, c0, w1, bf1, w2, bf2, w3, bf3)
    in_specs = [
        pl.BlockSpec((1, _M, 8 * _NH), lambda i: (0, 0, 0)),
        pl.BlockSpec((1, _M, 8 * _NH), lambda i: (1, 0, 0)),
    ] + [pl.BlockSpec(op.shape, lambda i: tuple([0] * op.ndim))
         for op in operands[2:]]
    return pl.pallas_call(
        _head_kernel,
        out_shape=(
            jax.ShapeDtypeStruct((_M, 512), jnp.float32),
            jax.ShapeDtypeStruct((2, _M, _HPAD), jnp.float32),
            jax.ShapeDtypeStruct((2, _M, _HPAD), jnp.float32),
        ),
        grid_spec=pltpu.PrefetchScalarGridSpec(
            num_scalar_prefetch=0,
            grid=(1,),
            in_specs=in_specs,
            out_specs=(
                pl.BlockSpec((_M, 512), lambda i: (0, 0)),
                pl.BlockSpec((2, _M, _HPAD), lambda i: (0, 0, 0)),
                pl.BlockSpec((2, _M, _HPAD), lambda i: (0, 0, 0)),
            ),
        ),
        compiler_params=pltpu.CompilerParams(
            dimension_semantics=("arbitrary",),
            vmem_limit_bytes=_VMEM,
        ),
    )(*operands)


def _build_input(node_positions, c_disp, gcn_output):
    act = jnp.maximum(gcn_output, 0.0)
    start0 = _MAP // 2 - _EGO // 2 + c_disp[0]
    start1 = _MAP // 2 - _EGO // 2 + c_disp[1]
    pos = lax.dynamic_slice(node_positions, (start0, start1, jnp.int32(0)),
                            (_EGO, _EGO, 2))
    feat = jnp.concatenate([act.reshape(_EGO, _EGO, _NCLS), pos], axis=2)
    nw = _EGO // _WIN
    feat = feat.reshape(nw, _WIN, nw, _WIN, _CHAN)
    feat = jnp.transpose(feat, (0, 2, 4, 1, 3))
    return feat.reshape(_M, _LSTM_IN)


def kernel(gcn_output, motion, c_disp, h, c, node_positions,
           w_l0, b_l0, w_l1, b_l1, w_fc1, b_fc1, w_fc2, b_fc2, w_fc3, b_fc3):
    c_disp_new = c_disp + motion.astype(jnp.int32)
    d = _build_input(node_positions, c_disp_new, gcn_output)
    d = jnp.pad(d, ((0, 0), (0, _IN_PAD - _LSTM_IN)))
    xh0 = jnp.concatenate([d, h[0]], axis=1)

    h0, c0, part = _run_lstm_pair(xh0, c[0], h[1], w_l0, b_l0, w_l1)
    out, h_stack, c_stack = _run_head(part, b_l1, c[1], h0, c0,
                                      w_fc1, b_fc1, w_fc2, b_fc2, w_fc3, b_fc3)

    out = out[:, :_OUT].reshape(_EGO * _EGO, _NCLS)
    new_state = {
        "c_disp": c_disp_new,
        "h": h_stack,
        "c": c_stack,
        "node_positions": node_positions,
    }
    return out, new_state


# speedup vs baseline: 1.2374x; 1.2374x over previous
import jax
import jax.numpy as jnp
from jax import lax
from jax.experimental import pallas as pl
from jax.experimental.pallas import tpu as pltpu

_MAP = 100
_WIN = 11
_EGO = 33
_NCLS = 4
_CHAN = _NCLS + 2
_LSTM_IN = _WIN * _WIN * _CHAN
_IN_PAD = 768
_HPAD = 1024
_NH = 512
_M = 9
_K0 = _IN_PAD + _HPAD
_K1 = 2 * _HPAD
_OUT = _WIN * _WIN * _NCLS
_VMEM = 100 * 1024 * 1024


def _gates(x_list, w_ref, b_ref, cp):
    pre = []
    for g in range(4):
        acc = b_ref[g, 0, 0].astype(jnp.float32)
        off = 0
        for x in x_list:
            k = x.shape[1]
            acc = acc + jnp.dot(x, w_ref[g, 0, off:off + k, :],
                                preferred_element_type=jnp.float32)
            off += k
        pre.append(acc)
    c_new = jax.nn.sigmoid(pre[1]) * cp + \
        jax.nn.sigmoid(pre[0]) * jnp.tanh(pre[2])
    h_new = jax.nn.sigmoid(pre[3]) * jnp.tanh(c_new)
    return h_new, c_new


def _layer0_kernel(xh_ref, w_ref, b_ref, c0p_ref, h0_ref, c0_ref):
    x = xh_ref[...].astype(jnp.bfloat16)
    h0, c0 = _gates([x], w_ref, b_ref, c0p_ref[...])
    h0_ref[...] = h0
    c0_ref[...] = c0


def _run_layer0(xh0, c0_prev, w0, b0):
    w0r = w0.reshape(4, 2, _K0, _NH)
    b0r = b0.reshape(4, 2, 1, _NH)
    return pl.pallas_call(
        _layer0_kernel,
        out_shape=(jax.ShapeDtypeStruct((_M, _HPAD), jnp.float32),
                   jax.ShapeDtypeStruct((_M, _HPAD), jnp.float32)),
        grid_spec=pltpu.PrefetchScalarGridSpec(
            num_scalar_prefetch=0,
            grid=(2,),
            in_specs=[
                pl.BlockSpec((_M, _K0), lambda n: (0, 0)),
                pl.BlockSpec((4, 1, _K0, _NH), lambda n: (0, n, 0, 0)),
                pl.BlockSpec((4, 1, 1, _NH), lambda n: (0, n, 0, 0)),
                pl.BlockSpec((_M, _NH), lambda n: (0, n)),
            ],
            out_specs=(pl.BlockSpec((_M, _NH), lambda n: (0, n)),
                       pl.BlockSpec((_M, _NH), lambda n: (0, n))),
        ),
        compiler_params=pltpu.CompilerParams(
            dimension_semantics=("parallel",),
            vmem_limit_bytes=_VMEM,
        ),
    )(xh0, w0r, b0r, c0_prev)


def _layer1_kernel(h0_ref, hp_ref, w_ref, b_ref, c0_ref, c1p_ref,
                   hs_ref, cs_ref):
    n = pl.program_id(0)
    h0b = h0_ref[...].astype(jnp.bfloat16)
    hpb = hp_ref[...].astype(jnp.bfloat16)
    h1, c1 = _gates([h0b, hpb], w_ref, b_ref, c1p_ref[...])
    hs_ref[0] = h0_ref[:, pl.ds(n * _NH, _NH)]
    hs_ref[1] = h1
    cs_ref[0] = c0_ref[...]
    cs_ref[1] = c1


def _run_layer1(h0, h1_prev, c0, c1_prev, w1, b1):
    w1r = w1.reshape(4, 2, _K1, _NH)
    b1r = b1.reshape(4, 2, 1, _NH)
    return pl.pallas_call(
        _layer1_kernel,
        out_shape=(jax.ShapeDtypeStruct((2, _M, _HPAD), jnp.float32),
                   jax.ShapeDtypeStruct((2, _M, _HPAD), jnp.float32)),
        grid_spec=pltpu.PrefetchScalarGridSpec(
            num_scalar_prefetch=0,
            grid=(2,),
            in_specs=[
                pl.BlockSpec((_M, _HPAD), lambda n: (0, 0)),
                pl.BlockSpec((_M, _HPAD), lambda n: (0, 0)),
                pl.BlockSpec((4, 1, _K1, _NH), lambda n: (0, n, 0, 0)),
                pl.BlockSpec((4, 1, 1, _NH), lambda n: (0, n, 0, 0)),
                pl.BlockSpec((_M, _NH), lambda n: (0, n)),
                pl.BlockSpec((_M, _NH), lambda n: (0, n)),
            ],
            out_specs=(
                pl.BlockSpec((2, _M, _NH), lambda n: (0, 0, n)),
                pl.BlockSpec((2, _M, _NH), lambda n: (0, 0, n)),
            ),
        ),
        compiler_params=pltpu.CompilerParams(
            dimension_semantics=("parallel",),
            vmem_limit_bytes=_VMEM,
        ),
    )(h0, h1_prev, w1r, b1r, c0, c1_prev)


def _head_kernel(h1_ref, w1_ref, bf1_ref, w2_ref, bf2_ref, w3_ref, bf3_ref,
                 out_ref):
    t = jnp.dot(h1_ref[0].astype(jnp.bfloat16), w1_ref[...],
                preferred_element_type=jnp.float32) + bf1_ref[...]
    t = jnp.maximum(t, 0.0)
    t = jnp.dot(t.astype(jnp.bfloat16), w2_ref[...],
                preferred_element_type=jnp.float32) + bf2_ref[...]
    t = jnp.maximum(t, 0.0)
    out_ref[...] = jnp.dot(t.astype(jnp.bfloat16), w3_ref[...],
                           preferred_element_type=jnp.float32) + bf3_ref[...]


def _run_head(h_stack, w1, bf1, w2, bf2, w3, bf3):
    operands = (h_stack, w1, bf1, w2, bf2, w3, bf3)
    in_specs = [pl.BlockSpec((1, _M, _HPAD), lambda i: (1, 0, 0))] + [
        pl.BlockSpec(op.shape, lambda i: tuple([0] * op.ndim))
        for op in operands[1:]]
    return pl.pallas_call(
        _head_kernel,
        out_shape=jax.ShapeDtypeStruct((_M, 512), jnp.float32),
        grid_spec=pltpu.PrefetchScalarGridSpec(
            num_scalar_prefetch=0,
            grid=(1,),
            in_specs=in_specs,
            out_specs=pl.BlockSpec((_M, 512), lambda i: (0, 0)),
        ),
        compiler_params=pltpu.CompilerParams(
            dimension_semantics=("arbitrary",),
            vmem_limit_bytes=_VMEM,
        ),
    )(*operands)


def _build_input(node_positions, c_disp, gcn_output):
    act = jnp.maximum(gcn_output, 0.0)
    start0 = _MAP // 2 - _EGO // 2 + c_disp[0]
    start1 = _MAP // 2 - _EGO // 2 + c_disp[1]
    pos = lax.dynamic_slice(node_positions, (start0, start1, jnp.int32(0)),
                            (_EGO, _EGO, 2))
    feat = jnp.concatenate([act.reshape(_EGO, _EGO, _NCLS), pos], axis=2)
    nw = _EGO // _WIN
    feat = feat.reshape(nw, _WIN, nw, _WIN, _CHAN)
    feat = jnp.transpose(feat, (0, 2, 4, 1, 3))
    return feat.reshape(_M, _LSTM_IN)


def kernel(gcn_output, motion, c_disp, h, c, node_positions,
           w_l0, b_l0, w_l1, b_l1, w_fc1, b_fc1, w_fc2, b_fc2, w_fc3, b_fc3):
    c_disp_new = c_disp + motion.astype(jnp.int32)
    d = _build_input(node_positions, c_disp_new, gcn_output)
    d = jnp.pad(d, ((0, 0), (0, _IN_PAD - _LSTM_IN)))
    xh0 = jnp.concatenate([d, h[0]], axis=1)

    h0, c0 = _run_layer0(xh0, c[0], w_l0, b_l0)
    h_stack, c_stack = _run_layer1(h0, h[1], c0, c[1], w_l1, b_l1)
    out = _run_head(h_stack, w_fc1, b_fc1, w_fc2, b_fc2, w_fc3, b_fc3)

    out = out[:, :_OUT].reshape(_EGO * _EGO, _NCLS)
    new_state = {
        "c_disp": c_disp_new,
        "h": h_stack,
        "c": c_stack,
        "node_positions": node_positions,
    }
    return out, new_state


# speedup vs baseline: 1.4992x; 1.2115x over previous
import jax
import jax.numpy as jnp
from jax import lax
from jax.experimental import pallas as pl
from jax.experimental.pallas import tpu as pltpu

_MAP = 100
_WIN = 11
_EGO = 33
_NCLS = 4
_CHAN = _NCLS + 2
_LSTM_IN = _WIN * _WIN * _CHAN
_IN_PAD = 768
_HPAD = 1024
_NH = 512
_M = 9
_K0 = _IN_PAD + _HPAD
_K1 = 2 * _HPAD
_OUT = _WIN * _WIN * _NCLS
_VMEM = 32 * 1024 * 1024


def _gates(x_list, w_refs, b_ref, cp):
    pre = []
    for g in range(4):
        acc = b_ref[g, 0, 0].astype(jnp.float32)
        off = 0
        for x in x_list:
            k = x.shape[1]
            acc = acc + jnp.dot(x, w_refs[g][0, 0, off:off + k, :],
                                preferred_element_type=jnp.float32)
            off += k
        pre.append(acc)
    c_new = jax.nn.sigmoid(pre[1]) * cp + \
        jax.nn.sigmoid(pre[0]) * jnp.tanh(pre[2])
    h_new = jax.nn.sigmoid(pre[3]) * jnp.tanh(c_new)
    return h_new, c_new


def _layer0_kernel(xh_ref, wi_ref, wf_ref, wg_ref, wo_ref, b_ref, c0p_ref,
                   h0_ref, c0_ref):
    x = xh_ref[...].astype(jnp.bfloat16)
    h0, c0 = _gates([x], (wi_ref, wf_ref, wg_ref, wo_ref), b_ref, c0p_ref[...])
    h0_ref[...] = h0
    c0_ref[...] = c0


def _run_layer0(xh0, c0_prev, w0, b0):
    w0r = w0.reshape(4, 2, _K0, _NH)
    b0r = b0.reshape(4, 2, 1, _NH)
    return pl.pallas_call(
        _layer0_kernel,
        out_shape=(jax.ShapeDtypeStruct((_M, _HPAD), jnp.float32),
                   jax.ShapeDtypeStruct((_M, _HPAD), jnp.float32)),
        grid_spec=pltpu.PrefetchScalarGridSpec(
            num_scalar_prefetch=0,
            grid=(2,),
            in_specs=[
                pl.BlockSpec((_M, _K0), lambda n: (0, 0)),
            ] + [pl.BlockSpec((1, 1, _K0, _NH), lambda n, g=g: (g, n, 0, 0))
                 for g in range(4)] + [
                pl.BlockSpec((4, 1, 1, _NH), lambda n: (0, n, 0, 0)),
                pl.BlockSpec((_M, _NH), lambda n: (0, n)),
            ],
            out_specs=(pl.BlockSpec((_M, _NH), lambda n: (0, n)),
                       pl.BlockSpec((_M, _NH), lambda n: (0, n))),
        ),
        compiler_params=pltpu.CompilerParams(
            dimension_semantics=("parallel",),
            vmem_limit_bytes=_VMEM,
        ),
    )(xh0, w0r, w0r, w0r, w0r, b0r, c0_prev)


def _layer1_kernel(h0_ref, hp_ref, wi_ref, wf_ref, wg_ref, wo_ref, b_ref,
                   c0_ref, c1p_ref, hs_ref, cs_ref):
    n = pl.program_id(0)
    h0b = h0_ref[...].astype(jnp.bfloat16)
    hpb = hp_ref[...].astype(jnp.bfloat16)
    h1, c1 = _gates([h0b, hpb], (wi_ref, wf_ref, wg_ref, wo_ref), b_ref,
                    c1p_ref[...])
    hs_ref[0] = h0_ref[:, pl.ds(n * _NH, _NH)]
    hs_ref[1] = h1
    cs_ref[0] = c0_ref[...]
    cs_ref[1] = c1


def _run_layer1(h0, h1_prev, c0, c1_prev, w1, b1):
    w1r = w1.reshape(4, 2, _K1, _NH)
    b1r = b1.reshape(4, 2, 1, _NH)
    return pl.pallas_call(
        _layer1_kernel,
        out_shape=(jax.ShapeDtypeStruct((2, _M, _HPAD), jnp.float32),
                   jax.ShapeDtypeStruct((2, _M, _HPAD), jnp.float32)),
        grid_spec=pltpu.PrefetchScalarGridSpec(
            num_scalar_prefetch=0,
            grid=(2,),
            in_specs=[
                pl.BlockSpec((_M, _HPAD), lambda n: (0, 0)),
                pl.BlockSpec((_M, _HPAD), lambda n: (0, 0)),
            ] + [pl.BlockSpec((1, 1, _K1, _NH), lambda n, g=g: (g, n, 0, 0))
                 for g in range(4)] + [
                pl.BlockSpec((4, 1, 1, _NH), lambda n: (0, n, 0, 0)),
                pl.BlockSpec((_M, _NH), lambda n: (0, n)),
                pl.BlockSpec((_M, _NH), lambda n: (0, n)),
            ],
            out_specs=(
                pl.BlockSpec((2, _M, _NH), lambda n: (0, 0, n)),
                pl.BlockSpec((2, _M, _NH), lambda n: (0, 0, n)),
            ),
        ),
        compiler_params=pltpu.CompilerParams(
            dimension_semantics=("parallel",),
            vmem_limit_bytes=_VMEM,
        ),
    )(h0, h1_prev, w1r, w1r, w1r, w1r, b1r, c0, c1_prev)


def _head_kernel(h1_ref, w1_ref, bf1_ref, w2_ref, bf2_ref, w3_ref, bf3_ref,
                 out_ref):
    t = jnp.dot(h1_ref[0].astype(jnp.bfloat16), w1_ref[...],
                preferred_element_type=jnp.float32) + bf1_ref[...]
    t = jnp.maximum(t, 0.0)
    t = jnp.dot(t.astype(jnp.bfloat16), w2_ref[...],
                preferred_element_type=jnp.float32) + bf2_ref[...]
    t = jnp.maximum(t, 0.0)
    out_ref[...] = jnp.dot(t.astype(jnp.bfloat16), w3_ref[...],
                           preferred_element_type=jnp.float32) + bf3_ref[...]


def _run_head(h_stack, w1, bf1, w2, bf2, w3, bf3):
    operands = (h_stack, w1, bf1, w2, bf2, w3, bf3)
    in_specs = [pl.BlockSpec((1, _M, _HPAD), lambda i: (1, 0, 0))] + [
        pl.BlockSpec(op.shape, lambda i: tuple([0] * op.ndim))
        for op in operands[1:]]
    return pl.pallas_call(
        _head_kernel,
        out_shape=jax.ShapeDtypeStruct((_M, 512), jnp.float32),
        grid_spec=pltpu.PrefetchScalarGridSpec(
            num_scalar_prefetch=0,
            grid=(1,),
            in_specs=in_specs,
            out_specs=pl.BlockSpec((_M, 512), lambda i: (0, 0)),
        ),
        compiler_params=pltpu.CompilerParams(
            dimension_semantics=("arbitrary",),
            vmem_limit_bytes=_VMEM,
        ),
    )(*operands)


def _build_input(node_positions, c_disp, gcn_output):
    act = jnp.maximum(gcn_output, 0.0)
    start0 = _MAP // 2 - _EGO // 2 + c_disp[0]
    start1 = _MAP // 2 - _EGO // 2 + c_disp[1]
    pos = lax.dynamic_slice(node_positions, (start0, start1, jnp.int32(0)),
                            (_EGO, _EGO, 2))
    feat = jnp.concatenate([act.reshape(_EGO, _EGO, _NCLS), pos], axis=2)
    nw = _EGO // _WIN
    feat = feat.reshape(nw, _WIN, nw, _WIN, _CHAN)
    feat = jnp.transpose(feat, (0, 2, 4, 1, 3))
    return feat.reshape(_M, _LSTM_IN)


def kernel(gcn_output, motion, c_disp, h, c, node_positions,
           w_l0, b_l0, w_l1, b_l1, w_fc1, b_fc1, w_fc2, b_fc2, w_fc3, b_fc3):
    c_disp_new = c_disp + motion.astype(jnp.int32)
    d = _build_input(node_positions, c_disp_new, gcn_output)
    d = jnp.pad(d, ((0, 0), (0, _IN_PAD - _LSTM_IN)))
    xh0 = jnp.concatenate([d, h[0]], axis=1)

    h0, c0 = _run_layer0(xh0, c[0], w_l0, b_l0)
    h_stack, c_stack = _run_layer1(h0, h[1], c0, c[1], w_l1, b_l1)
    out = _run_head(h_stack, w_fc1, b_fc1, w_fc2, b_fc2, w_fc3, b_fc3)

    out = out[:, :_OUT].reshape(_EGO * _EGO, _NCLS)
    new_state = {
        "c_disp": c_disp_new,
        "h": h_stack,
        "c": c_stack,
        "node_positions": node_positions,
    }
    return out, new_state


# speedup vs baseline: 1.5057x; 1.0043x over previous
import jax
import jax.numpy as jnp
from jax import lax
from jax.experimental import pallas as pl
from jax.experimental.pallas import tpu as pltpu

_MAP = 100
_WIN = 11
_EGO = 33
_NCLS = 4
_CHAN = _NCLS + 2
_LSTM_IN = _WIN * _WIN * _CHAN
_IN_PAD = 768
_HPAD = 1024
_NH = 512
_M = 9
_K0 = _IN_PAD + _HPAD
_K1 = 2 * _HPAD
_OUT = _WIN * _WIN * _NCLS
_VMEM = 32 * 1024 * 1024


def _gates(x_list, w_refs, b_ref, cp):
    n = pl.program_id(0)
    pre = []
    for g in range(4):
        acc = b_ref[0, pl.ds(g * 2 * _NH + n * _NH, _NH)].astype(jnp.float32)
        off = 0
        for x in x_list:
            k = x.shape[1]
            acc = acc + jnp.dot(x, w_refs[g][0, off:off + k, :],
                                preferred_element_type=jnp.float32)
            off += k
        pre.append(acc)
    c_new = jax.nn.sigmoid(pre[1]) * cp + \
        jax.nn.sigmoid(pre[0]) * jnp.tanh(pre[2])
    h_new = jax.nn.sigmoid(pre[3]) * jnp.tanh(c_new)
    return h_new, c_new


def _layer0_kernel(xh_ref, wi_ref, wf_ref, wg_ref, wo_ref, b_ref, c0p_ref,
                   h0_ref, c0_ref):
    x = xh_ref[...].astype(jnp.bfloat16)
    h0, c0 = _gates([x], (wi_ref, wf_ref, wg_ref, wo_ref), b_ref, c0p_ref[...])
    h0_ref[...] = h0
    c0_ref[...] = c0


def _run_layer0(xh0, c0_prev, w0, b0):
    return pl.pallas_call(
        _layer0_kernel,
        out_shape=(jax.ShapeDtypeStruct((_M, _HPAD), jnp.float32),
                   jax.ShapeDtypeStruct((_M, _HPAD), jnp.float32)),
        grid_spec=pltpu.PrefetchScalarGridSpec(
            num_scalar_prefetch=0,
            grid=(2,),
            in_specs=[
                pl.BlockSpec((_M, _K0), lambda n: (0, 0)),
            ] + [pl.BlockSpec((1, _K0, _NH), lambda n, g=g: (2 * g + n, 0, 0))
                 for g in range(4)] + [
                pl.BlockSpec((1, 8 * _NH), lambda n: (0, 0)),
                pl.BlockSpec((_M, _NH), lambda n: (0, n)),
            ],
            out_specs=(pl.BlockSpec((_M, _NH), lambda n: (0, n)),
                       pl.BlockSpec((_M, _NH), lambda n: (0, n))),
        ),
        compiler_params=pltpu.CompilerParams(
            dimension_semantics=("parallel",),
            vmem_limit_bytes=_VMEM,
        ),
    )(xh0, w0, w0, w0, w0, b0, c0_prev)


def _layer1_kernel(h0_ref, hp_ref, wi_ref, wf_ref, wg_ref, wo_ref, b_ref,
                   c0_ref, c1p_ref, hs_ref, cs_ref):
    n = pl.program_id(0)
    h0b = h0_ref[...].astype(jnp.bfloat16)
    hpb = hp_ref[...].astype(jnp.bfloat16)
    h1, c1 = _gates([h0b, hpb], (wi_ref, wf_ref, wg_ref, wo_ref), b_ref,
                    c1p_ref[...])
    hs_ref[0] = h0_ref[:, pl.ds(n * _NH, _NH)]
    hs_ref[1] = h1
    cs_ref[0] = c0_ref[...]
    cs_ref[1] = c1


def _run_layer1(h0, h1_prev, c0, c1_prev, w1, b1):
    return pl.pallas_call(
        _layer1_kernel,
        out_shape=(jax.ShapeDtypeStruct((2, _M, _HPAD), jnp.float32),
                   jax.ShapeDtypeStruct((2, _M, _HPAD), jnp.float32)),
        grid_spec=pltpu.PrefetchScalarGridSpec(
            num_scalar_prefetch=0,
            grid=(2,),
            in_specs=[
                pl.BlockSpec((_M, _HPAD), lambda n: (0, 0)),
                pl.BlockSpec((_M, _HPAD), lambda n: (0, 0)),
            ] + [pl.BlockSpec((1, _K1, _NH), lambda n, g=g: (2 * g + n, 0, 0))
                 for g in range(4)] + [
                pl.BlockSpec((1, 8 * _NH), lambda n: (0, 0)),
                pl.BlockSpec((_M, _NH), lambda n: (0, n)),
                pl.BlockSpec((_M, _NH), lambda n: (0, n)),
            ],
            out_specs=(
                pl.BlockSpec((2, _M, _NH), lambda n: (0, 0, n)),
                pl.BlockSpec((2, _M, _NH), lambda n: (0, 0, n)),
            ),
        ),
        compiler_params=pltpu.CompilerParams(
            dimension_semantics=("parallel",),
            vmem_limit_bytes=_VMEM,
        ),
    )(h0, h1_prev, w1, w1, w1, w1, b1, c0, c1_prev)


def _head_kernel(h1_ref, w1_ref, bf1_ref, w2_ref, bf2_ref, w3_ref, bf3_ref,
                 out_ref):
    t = jnp.dot(h1_ref[0].astype(jnp.bfloat16), w1_ref[...],
                preferred_element_type=jnp.float32) + bf1_ref[...]
    t = jnp.maximum(t, 0.0)
    t = jnp.dot(t.astype(jnp.bfloat16), w2_ref[...],
                preferred_element_type=jnp.float32) + bf2_ref[...]
    t = jnp.maximum(t, 0.0)
    out_ref[...] = jnp.dot(t.astype(jnp.bfloat16), w3_ref[...],
                           preferred_element_type=jnp.float32) + bf3_ref[...]


def _run_head(h_stack, w1, bf1, w2, bf2, w3, bf3):
    operands = (h_stack, w1, bf1, w2, bf2, w3, bf3)
    in_specs = [pl.BlockSpec((1, _M, _HPAD), lambda i: (1, 0, 0))] + [
        pl.BlockSpec(op.shape, lambda i: tuple([0] * op.ndim))
        for op in operands[1:]]
    return pl.pallas_call(
        _head_kernel,
        out_shape=jax.ShapeDtypeStruct((_M, 512), jnp.float32),
        grid_spec=pltpu.PrefetchScalarGridSpec(
            num_scalar_prefetch=0,
            grid=(1,),
            in_specs=in_specs,
            out_specs=pl.BlockSpec((_M, 512), lambda i: (0, 0)),
        ),
        compiler_params=pltpu.CompilerParams(
            dimension_semantics=("arbitrary",),
            vmem_limit_bytes=_VMEM,
        ),
    )(*operands)


def _build_input(node_positions, c_disp, gcn_output):
    act = jnp.maximum(gcn_output, 0.0)
    start0 = _MAP // 2 - _EGO // 2 + c_disp[0]
    start1 = _MAP // 2 - _EGO // 2 + c_disp[1]
    pos = lax.dynamic_slice(node_positions, (start0, start1, jnp.int32(0)),
                            (_EGO, _EGO, 2))
    feat = jnp.concatenate([act.reshape(_EGO, _EGO, _NCLS), pos], axis=2)
    nw = _EGO // _WIN
    feat = feat.reshape(nw, _WIN, nw, _WIN, _CHAN)
    feat = jnp.transpose(feat, (0, 2, 4, 1, 3))
    return feat.reshape(_M, _LSTM_IN)


def kernel(gcn_output, motion, c_disp, h, c, node_positions,
           w_l0, b_l0, w_l1, b_l1, w_fc1, b_fc1, w_fc2, b_fc2, w_fc3, b_fc3):
    c_disp_new = c_disp + motion.astype(jnp.int32)
    d = _build_input(node_positions, c_disp_new, gcn_output)
    d = jnp.pad(d, ((0, 0), (0, _IN_PAD - _LSTM_IN)))
    xh0 = jnp.concatenate([d, h[0]], axis=1)

    h0, c0 = _run_layer0(xh0, c[0], w_l0, b_l0)
    h_stack, c_stack = _run_layer1(h0, h[1], c0, c[1], w_l1, b_l1)
    out = _run_head(h_stack, w_fc1, b_fc1, w_fc2, b_fc2, w_fc3, b_fc3)

    out = out[:, :_OUT].reshape(_EGO * _EGO, _NCLS)
    new_state = {
        "c_disp": c_disp_new,
        "h": h_stack,
        "c": c_stack,
        "node_positions": node_positions,
    }
    return out, new_state
